# Initial kernel scaffold; baseline (speedup 1.0000x reference)
import jax
import jax.numpy as jnp
from jax.experimental import pallas as pl


def kernel(category_indices, table):
    raise NotImplementedError("write your pallas kernel here")


# speedup vs baseline: 4.9476x; 4.9476x over previous
import functools

import jax
import jax.numpy as jnp
from jax import lax
from jax.experimental import pallas as pl
from jax.experimental.pallas import tpu as pltpu
from jax.experimental.pallas import tpu_sc as plsc


def _make_gather(N, V, D, num_cores, num_subcores):
    NW = num_cores * num_subcores
    n_per_w = N // NW
    CH = 2048
    n_chunks = n_per_w // CH

    mesh = plsc.VectorSubcoreMesh(core_axis_name="c", subcore_axis_name="s")

    @functools.partial(
        pl.kernel,
        mesh=mesh,
        compiler_params=pltpu.CompilerParams(use_tc_tiling_on_sc=False),
        out_type=jax.ShapeDtypeStruct((N, D), jnp.float32),
        scratch_types=[
            pltpu.VMEM((CH,), jnp.int32),
            pltpu.VMEM((CH, D), jnp.float32),
            pltpu.SemaphoreType.DMA,
        ],
    )
    def gather_k(idx_hbm, table_hbm, out_hbm, idx_v, rows_v, sem):
        wid = lax.axis_index("s") * num_cores + lax.axis_index("c")
        base_w = wid * n_per_w

        def body(c, carry):
            base = base_w + c * CH
            pltpu.sync_copy(idx_hbm.at[pl.ds(base, CH)], idx_v)
            pltpu.async_copy(table_hbm.at[idx_v], rows_v, sem).wait()
            pltpu.sync_copy(rows_v, out_hbm.at[pl.ds(base, CH)])
            return carry

        lax.fori_loop(0, n_chunks, body, 0)

    return gather_k


def kernel(category_indices, table):
    B, S = category_indices.shape
    V, D = table.shape
    N = B * S
    idx = category_indices.reshape(N).astype(jnp.int32)

    info = plsc.get_sparse_core_info()
    gather_k = _make_gather(N, V, D, info.num_cores, info.num_subcores)
    out = gather_k(idx, table)
    return out.reshape(B, S, D)
